# baseline (speedup 1.0000x reference)
import jax
import jax.numpy as jnp
from jax import lax
from jax.experimental import pallas as pl
from jax.experimental.pallas import tpu as pltpu

_SEQ_BLOCK = 512


def _body(pe_ref, out_ref):
    out_ref[...] = pe_ref[...][None]


def kernel(inputs, position_embeddings):
    batch, seq, width = inputs.shape
    pe = position_embeddings[:seq, :]
    n_seq_blocks = seq // _SEQ_BLOCK
    out = pl.pallas_call(
        _body,
        grid=(n_seq_blocks, batch),
        in_specs=[pl.BlockSpec((_SEQ_BLOCK, width), lambda i, b: (i, 0))],
        out_specs=pl.BlockSpec((1, _SEQ_BLOCK, width), lambda i, b: (b, i, 0)),
        out_shape=jax.ShapeDtypeStruct((batch, seq, width), jnp.float32),
    )(pe)
    return out


# speedup vs baseline: 1.2953x; 1.2953x over previous
import jax
import jax.numpy as jnp
from jax import lax
from jax.experimental import pallas as pl
from jax.experimental.pallas import tpu as pltpu

_SEQ_BLOCK = 512


def _make_body(batch):
    def body(pe_ref, out_ref, sem):
        i = pl.program_id(0)
        copies = [
            pltpu.make_async_copy(
                pe_ref, out_ref.at[b, pl.ds(i * _SEQ_BLOCK, _SEQ_BLOCK)], sem)
            for b in range(batch)
        ]
        for c in copies:
            c.start()
        for c in copies:
            c.wait()
    return body


def kernel(inputs, position_embeddings):
    batch, seq, width = inputs.shape
    pe = position_embeddings[:seq, :]
    n_seq_blocks = seq // _SEQ_BLOCK
    out = pl.pallas_call(
        _make_body(batch),
        grid=(n_seq_blocks,),
        in_specs=[pl.BlockSpec((_SEQ_BLOCK, width), lambda i: (i, 0))],
        out_specs=pl.BlockSpec(memory_space=pl.ANY),
        out_shape=jax.ShapeDtypeStruct((batch, seq, width), jnp.float32),
        scratch_shapes=[pltpu.SemaphoreType.DMA],
    )(pe)
    return out


# speedup vs baseline: 1.4298x; 1.1039x over previous
import jax
import jax.numpy as jnp
from jax import lax
from jax.experimental import pallas as pl
from jax.experimental.pallas import tpu as pltpu

_SEQ_BLOCK = 1024


def _make_body(batch):
    def body(pe_ref, out_ref, sem):
        i = pl.program_id(0)
        copies = [
            pltpu.make_async_copy(
                pe_ref, out_ref.at[b, pl.ds(i * _SEQ_BLOCK, _SEQ_BLOCK)], sem)
            for b in range(batch)
        ]
        for c in copies:
            c.start()
        for c in copies:
            c.wait()
    return body


def kernel(inputs, position_embeddings):
    batch, seq, width = inputs.shape
    pe = position_embeddings[:seq, :]
    n_seq_blocks = seq // _SEQ_BLOCK
    out = pl.pallas_call(
        _make_body(batch),
        grid=(n_seq_blocks,),
        in_specs=[pl.BlockSpec((_SEQ_BLOCK, width), lambda i: (i, 0))],
        out_specs=pl.BlockSpec(memory_space=pl.ANY),
        out_shape=jax.ShapeDtypeStruct((batch, seq, width), jnp.float32),
        scratch_shapes=[pltpu.SemaphoreType.DMA],
    )(pe)
    return out
